# baseline (speedup 1.0000x reference)
import functools

import jax
import jax.numpy as jnp
from jax import lax
from jax.experimental import pallas as pl
from jax.experimental.pallas import tpu as pltpu
from jax.experimental.pallas import tpu_sc as plsc

B = 16
MAX_LEN = 4096
D = 1024
NW = 32
ROWS_PER_W = (B * MAX_LEN) // NW
CHUNK = 128
ZCHUNK = 64


def _pad_body(flat_hbm, params_hbm, zeros_hbm, out_hbm, pvec, zbuf, sem):
    wid = lax.axis_index("s") * 2 + lax.axis_index("c")

    pltpu.sync_copy(params_hbm.at[pl.ds(wid * 16, 16)], pvec)
    pltpu.sync_copy(zeros_hbm, zbuf)

    pv = pvec[...]
    start = pv[0]
    valid = pv[1]
    outbase = wid * ROWS_PER_W

    nf = valid // CHUNK

    def copy_body(i, carry):
        pltpu.async_copy(
            flat_hbm.at[pl.ds((start + i * CHUNK) * D, CHUNK * D)],
            out_hbm.at[pl.ds((outbase + i * CHUNK) * D, CHUNK * D)],
            sem,
        ).wait()
        return carry

    lax.fori_loop(0, nf, copy_body, 0)

    r = valid - nf * CHUNK
    src_off = start + nf * CHUNK
    dst_off = outbase + nf * CHUNK
    consumed = jnp.int32(0)
    for k in (6, 5, 4, 3, 2, 1, 0):
        sz = 1 << k
        take = (r >> k) & 1
        s_off = src_off + consumed
        d_off = dst_off + consumed

        @pl.when(take == 1)
        def _copy_rem(s_off=s_off, d_off=d_off, sz=sz):
            pltpu.async_copy(
                flat_hbm.at[pl.ds(s_off * D, sz * D)],
                out_hbm.at[pl.ds(d_off * D, sz * D)],
                sem,
            ).wait()

        consumed = consumed + take * sz

    zoff = outbase + valid
    zrem = ROWS_PER_W - valid
    nz = zrem // ZCHUNK

    def zero_body(i, carry):
        pltpu.async_copy(
            zbuf, out_hbm.at[pl.ds((zoff + i * ZCHUNK) * D, ZCHUNK * D)], sem
        ).wait()
        return carry

    lax.fori_loop(0, nz, zero_body, 0)

    zr = zrem - nz * ZCHUNK
    zo = zoff + nz * ZCHUNK
    zconsumed = jnp.int32(0)
    for k in (5, 4, 3, 2, 1, 0):
        sz = 1 << k
        take = (zr >> k) & 1
        d_off = zo + zconsumed

        @pl.when(take == 1)
        def _zero_rem(d_off=d_off, sz=sz):
            pltpu.async_copy(
                zbuf.at[pl.ds(0, sz * D)],
                out_hbm.at[pl.ds(d_off * D, sz * D)],
                sem,
            ).wait()

        zconsumed = zconsumed + take * sz


_pad_kernel = functools.partial(
    pl.kernel,
    out_type=jax.ShapeDtypeStruct((B * MAX_LEN * D,), jnp.float32),
    mesh=plsc.VectorSubcoreMesh(core_axis_name="c", subcore_axis_name="s"),
    scratch_types=[
        pltpu.VMEM((16,), jnp.int32),
        pltpu.VMEM((ZCHUNK * D,), jnp.float32),
        pltpu.SemaphoreType.DMA,
    ],
)(_pad_body)


def kernel(flat, cu_seqlens):
    cu = cu_seqlens.astype(jnp.int32)
    lens32 = cu[1:] - cu[:-1]

    w = jnp.arange(NW, dtype=jnp.int32)
    b = w // 2
    t0 = (w % 2) * ROWS_PER_W
    starts = cu[:-1][b] + t0
    valids = jnp.clip(lens32[b] - t0, 0, ROWS_PER_W)
    params = jnp.zeros((NW, 16), jnp.int32)
    params = params.at[:, 0].set(starts).at[:, 1].set(valids)

    zeros = jnp.zeros((ZCHUNK * D,), jnp.float32)
    out = _pad_kernel(flat.reshape(-1), params.reshape(-1), zeros)
    padded = out.reshape(B, MAX_LEN, D)
    lens = lens32.astype(jnp.int64)
    return padded, lens
